# Initial kernel scaffold; baseline (speedup 1.0000x reference)
import jax
import jax.numpy as jnp
from jax.experimental import pallas as pl


def kernel(logits, val_freqs):
    raise NotImplementedError("write your pallas kernel here")


# speedup vs baseline: 418.3392x; 418.3392x over previous
import functools

import jax
import jax.numpy as jnp
from jax import lax
from jax.experimental import pallas as pl
from jax.experimental.pallas import tpu as pltpu
from jax.experimental.pallas import tpu_sc as plsc

NUM_BINS = 15
NUM_CLASSES = 19
LANES = 16

_GATHER_DNUMS = lax.GatherDimensionNumbers(
    offset_dims=(), collapsed_slice_dims=(0,), start_index_map=(0,))


def _take16(row, idx):
    return lax.gather(
        row, idx[:, None], _GATHER_DNUMS, (1,),
        mode=lax.GatherScatterMode.PROMISE_IN_BOUNDS)

B, H, W = 4, 512, 512
PIX_PER_BATCH = H * W
NW = 32
WORKERS_PER_BATCH = NW // B
PIX_PER_WORKER = PIX_PER_BATCH // WORKERS_PER_BATCH
CHUNK = 1024
NCHUNKS = PIX_PER_WORKER // CHUNK
GROUPS = CHUNK // LANES


def _sc_body(x_hbm, tab_hbm, out_hbm, in_v, out_v, tab_v):
    info = plsc.get_sparse_core_info()
    nc = info.num_cores
    wid = lax.axis_index("s") * nc + lax.axis_index("c")
    batch = wid // WORKERS_PER_BATCH
    col_base = (wid % WORKERS_PER_BATCH) * PIX_PER_WORKER

    pltpu.sync_copy(tab_hbm, tab_v)

    def chunk_body(ch, _):
        col0 = col_base + ch * CHUNK
        pltpu.sync_copy(x_hbm.at[batch, :, pl.ds(col0, CHUNK)], in_v)

        def group_body(g, _):
            sl = pl.ds(g * LANES, LANES)
            v = [in_v[c, sl] for c in range(NUM_CLASSES)]
            m = v[0]
            for c in range(1, NUM_CLASSES):
                m = jnp.maximum(m, v[c])
            e = [jnp.exp(v[c] - m) for c in range(NUM_CLASSES)]
            s = e[0]
            for c in range(1, NUM_CLASSES):
                s = s + e[c]
            r15 = jnp.float32(NUM_BINS) / s
            g_vals = []
            for c in range(NUM_CLASSES):
                bin_c = jnp.minimum(
                    (e[c] * r15).astype(jnp.int32), NUM_BINS - 1)
                row = tab_v[pl.ds(c * LANES, LANES)]
                g_vals.append(_take16(row, bin_c))
            t = g_vals[0]
            for c in range(1, NUM_CLASSES):
                t = t + g_vals[c]
            rn = jnp.float32(1.0) / t
            for c in range(NUM_CLASSES):
                out_v[c, sl] = g_vals[c] * rn
            return 0

        lax.fori_loop(0, GROUPS, group_body, 0)
        pltpu.sync_copy(out_v, out_hbm.at[batch, :, pl.ds(col0, CHUNK)])
        return 0

    lax.fori_loop(0, NCHUNKS, chunk_body, 0)


@jax.jit
def kernel(logits, val_freqs):
    x = logits.reshape(B, NUM_CLASSES, PIX_PER_BATCH)
    tab = jnp.pad(val_freqs, ((0, 0), (0, LANES - NUM_BINS))).reshape(-1)
    run = pl.kernel(
        _sc_body,
        out_type=jax.ShapeDtypeStruct((B, NUM_CLASSES, PIX_PER_BATCH),
                                      jnp.float32),
        mesh=plsc.VectorSubcoreMesh(core_axis_name="c", subcore_axis_name="s"),
        scratch_types=[
            pltpu.VMEM((NUM_CLASSES, CHUNK), jnp.float32),
            pltpu.VMEM((NUM_CLASSES, CHUNK), jnp.float32),
            pltpu.VMEM((NUM_CLASSES * LANES,), jnp.float32),
        ],
    )
    out = run(x, tab)
    return out.reshape(B, NUM_CLASSES, H, W)


# speedup vs baseline: 625.2313x; 1.4946x over previous
import functools

import jax
import jax.numpy as jnp
from jax import lax
from jax.experimental import pallas as pl
from jax.experimental.pallas import tpu as pltpu
from jax.experimental.pallas import tpu_sc as plsc

NUM_BINS = 15
NUM_CLASSES = 19
LANES = 16

B, H, W = 4, 512, 512
PIX_PER_BATCH = H * W
NW = 32
WORKERS_PER_BATCH = NW // B
PIX_PER_WORKER = PIX_PER_BATCH // WORKERS_PER_BATCH
CHUNK = 1024
NCHUNKS = PIX_PER_WORKER // CHUNK
GROUPS = CHUNK // LANES

_GATHER_DNUMS = lax.GatherDimensionNumbers(
    offset_dims=(), collapsed_slice_dims=(0,), start_index_map=(0,))


def _take16(row, idx):
    return lax.gather(
        row, idx[:, None], _GATHER_DNUMS, (1,),
        mode=lax.GatherScatterMode.PROMISE_IN_BOUNDS)


def _compute_tile(in_v, out_v, tab_v):
    @plsc.parallel_loop(0, GROUPS, unroll=2)
    def _(g):
        sl = pl.ds(g * LANES, LANES)
        v = [in_v[c, sl] for c in range(NUM_CLASSES)]
        m = v[0]
        for c in range(1, NUM_CLASSES):
            m = jnp.maximum(m, v[c])
        e = [jnp.exp(v[c] - m) for c in range(NUM_CLASSES)]
        s = e[0]
        for c in range(1, NUM_CLASSES):
            s = s + e[c]
        r15 = jnp.float32(NUM_BINS) / s
        g_vals = []
        for c in range(NUM_CLASSES):
            bin_c = (e[c] * r15).astype(jnp.int32)
            row = tab_v[pl.ds(c * LANES, LANES)]
            g_vals.append(_take16(row, bin_c))
        t = g_vals[0]
        for c in range(1, NUM_CLASSES):
            t = t + g_vals[c]
        rn = jnp.float32(1.0) / t
        for c in range(NUM_CLASSES):
            out_v[c, sl] = g_vals[c] * rn


def _sc_body(x_hbm, tab_hbm, out_hbm, in0, in1, out0, out1, tab_v,
             si0, si1, so0, so1):
    info = plsc.get_sparse_core_info()
    nc = info.num_cores
    wid = lax.axis_index("s") * nc + lax.axis_index("c")
    batch = wid // WORKERS_PER_BATCH
    col_base = (wid % WORKERS_PER_BATCH) * PIX_PER_WORKER

    pltpu.sync_copy(tab_hbm, tab_v)

    ins = (in0, in1)
    outs = (out0, out1)
    sis = (si0, si1)
    sos = (so0, so1)

    def src(ch):
        return x_hbm.at[batch, :, pl.ds(col_base + ch * CHUNK, CHUNK)]

    def dst(ch):
        return out_hbm.at[batch, :, pl.ds(col_base + ch * CHUNK, CHUNK)]

    pltpu.async_copy(src(0), in0, si0)
    pltpu.async_copy(src(1), in1, si1)

    def ring_body(k, _):
        for b in range(2):
            ch = 2 * k + b
            pltpu.make_async_copy(src(ch), ins[b], sis[b]).wait()

            @pl.when(k >= 1)
            def _():
                pltpu.make_async_copy(outs[b], dst(ch - 2), sos[b]).wait()

            _compute_tile(ins[b], outs[b], tab_v)
            pltpu.async_copy(outs[b], dst(ch), sos[b])

            @pl.when(k < NCHUNKS // 2 - 1)
            def _():
                pltpu.async_copy(src(ch + 2), ins[b], sis[b])
        return 0

    lax.fori_loop(0, NCHUNKS // 2, ring_body, 0)
    last = NCHUNKS - 2
    pltpu.make_async_copy(out0, dst(last), so0).wait()
    pltpu.make_async_copy(out1, dst(last + 1), so1).wait()


@jax.jit
def kernel(logits, val_freqs):
    x = logits.reshape(B, NUM_CLASSES, PIX_PER_BATCH)
    tab = jnp.concatenate(
        [val_freqs, val_freqs[:, NUM_BINS - 1:]], axis=1).reshape(-1)
    run = pl.kernel(
        _sc_body,
        out_type=jax.ShapeDtypeStruct((B, NUM_CLASSES, PIX_PER_BATCH),
                                      jnp.float32),
        mesh=plsc.VectorSubcoreMesh(core_axis_name="c", subcore_axis_name="s"),
        scratch_types=[
            pltpu.VMEM((NUM_CLASSES, CHUNK), jnp.float32),
            pltpu.VMEM((NUM_CLASSES, CHUNK), jnp.float32),
            pltpu.VMEM((NUM_CLASSES, CHUNK), jnp.float32),
            pltpu.VMEM((NUM_CLASSES, CHUNK), jnp.float32),
            pltpu.VMEM((NUM_CLASSES * LANES,), jnp.float32),
            pltpu.SemaphoreType.DMA,
            pltpu.SemaphoreType.DMA,
            pltpu.SemaphoreType.DMA,
            pltpu.SemaphoreType.DMA,
        ],
    )
    out = run(x, tab)
    return out.reshape(B, NUM_CLASSES, H, W)
